# baseline (speedup 1.0000x reference)
import functools

import jax
import jax.numpy as jnp
from jax import lax
from jax.experimental import pallas as pl
from jax.experimental.pallas import tpu as pltpu
from jax.experimental.pallas import tpu_sc as plsc

N = 10000
E = 320000
D = 128
H = 64
G = 128

NC = 2
NS = 16
NW = NC * NS
NPAD = 10240
EPC = E // NC
EPT = E // NW
K = 128
NCH = -(-EPT // K)
EPAD = NCH * K - EPT
DEGW = 16
RPT = NPAD // NS

BN = 512
NB = NPAD // BN

_sc_mesh = plsc.VectorSubcoreMesh(core_axis_name="c", subcore_axis_name="s")


WIN = 8
NBUF = 7
AHEAD = 3


@functools.partial(
    pl.kernel,
    out_type=jax.ShapeDtypeStruct((NC * NPAD, DEGW), jnp.float32),
    mesh=_sc_mesh,
    scratch_types=[
        pltpu.VMEM((NCH, K), jnp.int32),
        pltpu.VMEM((K, DEGW), jnp.float32),
        pltpu.SemaphoreType.DMA,
        pltpu.VMEM_SHARED((NPAD, DEGW), jnp.float32),
    ],
    compiler_params=pltpu.CompilerParams(use_tc_tiling_on_sc=False),
)
def _deg_kernel(dst_hbm, z16_hbm, out_hbm, dst2_v, ones_v, ssem, acc_sh):
    c = lax.axis_index("c")
    s = lax.axis_index("s")
    w = c * NS + s
    pltpu.sync_copy(dst_hbm.at[w], dst2_v)

    def fill(i, carry):
        ones_v[i, :] = jnp.ones((16,), jnp.float32)
        return carry

    lax.fori_loop(0, K, fill, 0)
    pltpu.sync_copy(z16_hbm.at[pl.ds(s * RPT, RPT)], acc_sh.at[pl.ds(s * RPT, RPT)])
    plsc.subcore_barrier()

    for b in range(WIN):
        pltpu.async_copy(ones_v, acc_sh.at[dst2_v.at[b]], ssem, add=True)

    def chunk(i, carry):
        pltpu.make_async_copy(ones_v, acc_sh.at[dst2_v.at[0]], ssem).wait()
        pltpu.async_copy(ones_v, acc_sh.at[dst2_v.at[i + WIN]], ssem, add=True)
        return carry

    lax.fori_loop(0, NCH - WIN, chunk, 0)
    for b in range(WIN):
        pltpu.make_async_copy(ones_v, acc_sh.at[dst2_v.at[0]], ssem).wait()
    plsc.subcore_barrier()
    pltpu.sync_copy(acc_sh.at[pl.ds(s * RPT, RPT)],
                    out_hbm.at[pl.ds(c * NPAD + s * RPT, RPT)])


@functools.partial(
    pl.kernel,
    out_type=jax.ShapeDtypeStruct((NC * NPAD, H), jnp.float32),
    mesh=_sc_mesh,
    scratch_types=[
        pltpu.VMEM((NCH, K), jnp.int32),
        pltpu.VMEM((NCH, K), jnp.int32),
        pltpu.VMEM((NBUF, K, H), jnp.float32),
        pltpu.SemaphoreType.DMA,
    ] + [pltpu.SemaphoreType.DMA] * NBUF + [
        pltpu.VMEM_SHARED((NPAD, H), jnp.float32),
    ],
    compiler_params=pltpu.CompilerParams(use_tc_tiling_on_sc=False),
)
def _msg_kernel(m_hbm, src_hbm, dst_hbm, out_hbm, src2_v, dst2_v, rows_v, gsem,
                *rest):
    ssems = list(rest[:NBUF])
    acc_sh = rest[NBUF]
    c = lax.axis_index("c")
    s = lax.axis_index("s")
    w = c * NS + s
    pltpu.sync_copy(src_hbm.at[w], src2_v)
    pltpu.sync_copy(dst_hbm.at[w], dst2_v)
    for b in range(AHEAD):
        pltpu.async_copy(m_hbm.at[src2_v.at[b]], rows_v.at[b], gsem)
    pltpu.sync_copy(m_hbm.at[pl.ds(s * RPT, RPT)], acc_sh.at[pl.ds(s * RPT, RPT)])
    plsc.subcore_barrier()

    def step(j, carry, tail):
        for b in (range(NCH % NBUF) if tail else range(NBUF)):
            i = j * NBUF + b
            pltpu.make_async_copy(m_hbm.at[src2_v.at[0]], rows_v.at[b],
                                  gsem).wait()
            pltpu.async_copy(rows_v.at[b], acc_sh.at[dst2_v.at[i]],
                             ssems[b], add=True)
            if not tail:
                bn = (b + AHEAD) % NBUF

                @pl.when(i + AHEAD < NCH)
                def _():
                    @pl.when(i >= NBUF - AHEAD)
                    def _():
                        pltpu.make_async_copy(rows_v.at[bn],
                                              acc_sh.at[dst2_v.at[0]],
                                              ssems[bn]).wait()

                    pltpu.async_copy(m_hbm.at[src2_v.at[i + AHEAD]],
                                     rows_v.at[bn], gsem)
        return carry

    nfull = NCH // NBUF
    lax.fori_loop(0, nfull, functools.partial(step, tail=False), 0)
    if NCH % NBUF:
        step(nfull, 0, tail=True)
    for b in range(NBUF):
        pltpu.make_async_copy(rows_v.at[0], acc_sh.at[dst2_v.at[0]],
                              ssems[b]).wait()
    plsc.subcore_barrier()
    pltpu.sync_copy(acc_sh.at[pl.ds(s * RPT, RPT)],
                    out_hbm.at[pl.ds(c * NPAD + s * RPT, RPT)])


def _dinv(deg_ref):
    deg = 1.0 + deg_ref[0, :, 0:1] + deg_ref[1, :, 0:1]
    return lax.rsqrt(deg)


def _tc_xw1_body(x_ref, w1_ref, o_ref):
    o_ref[...] = jnp.dot(x_ref[...], w1_ref[...],
                         preferred_element_type=jnp.float32)


def _tc_m1_body(deg_ref, xw1_ref, m1_ref):
    m1_ref[...] = xw1_ref[...] * _dinv(deg_ref)


def _tc_mid_body(acc_ref, m1_ref, deg_ref, b1_ref, w2_ref, m2_ref):
    dinv = _dinv(deg_ref)
    tot = acc_ref[0] + acc_ref[1] - m1_ref[...]
    h1 = jnp.maximum(tot * dinv + b1_ref[...], 0.0)
    m2_ref[...] = jnp.dot(h1, w2_ref[...],
                          preferred_element_type=jnp.float32) * dinv


def _tc_final_body(acc_ref, m2_ref, deg_ref, b2_ref, batch_ref, w3_ref,
                   b3_ref, w4_ref, b4_ref, psum_ref, pcnt_ref, o_ref):
    i = pl.program_id(0)
    dinv = _dinv(deg_ref)
    tot = acc_ref[0] + acc_ref[1] - m2_ref[...]
    h2 = jnp.maximum(tot * dinv + b2_ref[...], 0.0)
    b = batch_ref[0, 0, :]
    onehot = (b[:, None] == lax.broadcasted_iota(jnp.int32, (BN, G), 1))
    onehot = onehot.astype(jnp.float32)
    ps = lax.dot_general(onehot, h2, (((0,), (0,)), ((), ())),
                         preferred_element_type=jnp.float32)
    cs = lax.dot_general(onehot, jnp.ones((BN, H), jnp.float32),
                         (((0,), (0,)), ((), ())),
                         preferred_element_type=jnp.float32)

    @pl.when(i == 0)
    def _():
        psum_ref[...] = jnp.zeros_like(psum_ref)
        pcnt_ref[...] = jnp.zeros_like(pcnt_ref)

    psum_ref[...] += ps
    pcnt_ref[...] += cs

    @pl.when(i == NB - 1)
    def _():
        pooled = psum_ref[...] / jnp.maximum(pcnt_ref[...], 1.0)
        t = jnp.maximum(jnp.dot(pooled, w3_ref[...],
                                preferred_element_type=jnp.float32)
                        + b3_ref[...], 0.0)
        o_ref[...] = jnp.dot(t, w4_ref[...],
                             preferred_element_type=jnp.float32) + b4_ref[...]


def _full(shape):
    nd = len(shape)
    return pl.BlockSpec(shape, lambda i: (0,) * nd)


def kernel(x, edge_index, batch, W1, b1, W2, b2, W3, b3, W4, b4):
    x = x.astype(jnp.float32)
    src = jnp.pad(edge_index[0].reshape(NW, EPT),
                  ((0, 0), (0, EPAD))).reshape(NW, NCH, K)
    dst = jnp.pad(edge_index[1].reshape(NW, EPT), ((0, 0), (0, EPAD)),
                  constant_values=NPAD - 1).reshape(NW, NCH, K)
    xp = jnp.pad(x, ((0, NPAD - N), (0, 0)))
    z16 = jnp.zeros((NPAD, DEGW), jnp.float32)

    xw1 = pl.pallas_call(
        _tc_xw1_body,
        grid=(NB,),
        in_specs=[
            pl.BlockSpec((BN, D), lambda i: (i, 0)),
            _full((D, H)),
        ],
        out_specs=pl.BlockSpec((BN, H), lambda i: (i, 0)),
        out_shape=jax.ShapeDtypeStruct((NPAD, H), jnp.float32),
    )(xp, W1)

    degparts = _deg_kernel(dst, z16).reshape(NC, NPAD, DEGW)

    m1 = pl.pallas_call(
        _tc_m1_body,
        grid=(NB,),
        in_specs=[
            pl.BlockSpec((NC, BN, DEGW), lambda i: (0, i, 0)),
            pl.BlockSpec((BN, H), lambda i: (i, 0)),
        ],
        out_specs=pl.BlockSpec((BN, H), lambda i: (i, 0)),
        out_shape=jax.ShapeDtypeStruct((NPAD, H), jnp.float32),
    )(degparts, xw1)

    acc1 = _msg_kernel(m1, src, dst).reshape(NC, NPAD, H)

    m2 = pl.pallas_call(
        _tc_mid_body,
        grid=(NB,),
        in_specs=[
            pl.BlockSpec((NC, BN, H), lambda i: (0, i, 0)),
            pl.BlockSpec((BN, H), lambda i: (i, 0)),
            pl.BlockSpec((NC, BN, DEGW), lambda i: (0, i, 0)),
            _full((1, H)),
            _full((H, H)),
        ],
        out_specs=pl.BlockSpec((BN, H), lambda i: (i, 0)),
        out_shape=jax.ShapeDtypeStruct((NPAD, H), jnp.float32),
    )(acc1, m1, degparts, b1.reshape(1, H), W2)

    acc2 = _msg_kernel(m2, src, dst).reshape(NC, NPAD, H)

    batchp = jnp.pad(batch, (0, NPAD - N), constant_values=G)
    batchp = batchp.reshape(NB, 1, BN)
    w4p = jnp.pad(W4, ((0, 0), (0, 7)))

    _, _, o8 = pl.pallas_call(
        _tc_final_body,
        grid=(NB,),
        in_specs=[
            pl.BlockSpec((NC, BN, H), lambda i: (0, i, 0)),
            pl.BlockSpec((BN, H), lambda i: (i, 0)),
            pl.BlockSpec((NC, BN, DEGW), lambda i: (0, i, 0)),
            _full((1, H)),
            pl.BlockSpec((1, 1, BN), lambda i: (i, 0, 0)),
            _full((H, H)),
            _full((1, H)),
            _full((H, 8)),
            _full((1, 1)),
        ],
        out_specs=[_full((G, H)), _full((G, H)), _full((G, 8))],
        out_shape=[
            jax.ShapeDtypeStruct((G, H), jnp.float32),
            jax.ShapeDtypeStruct((G, H), jnp.float32),
            jax.ShapeDtypeStruct((G, 8), jnp.float32),
        ],
    )(acc2, m2, degparts, b2.reshape(1, H), batchp, W3,
      b3.reshape(1, H), w4p, b4.reshape(1, 1))

    return o8[:, 0]


# speedup vs baseline: 1.5618x; 1.5618x over previous
import functools

import jax
import jax.numpy as jnp
from jax import lax
from jax.experimental import pallas as pl
from jax.experimental.pallas import tpu as pltpu
from jax.experimental.pallas import tpu_sc as plsc

N = 10000
E = 320000
D = 128
H = 64
G = 128

NC = 2
NS = 16
NW = NC * NS
NPAD = 10240
EPC = E // NC
EPT = E // NW
K = 80
NCH = -(-EPT // K)
EPAD = NCH * K - EPT
DEGW = 16
RPT = NPAD // NS

BN = 512
NB = NPAD // BN

_sc_mesh = plsc.VectorSubcoreMesh(core_axis_name="c", subcore_axis_name="s")


WIN = 8
NBUF = 10
AHEAD = 5


@functools.partial(
    pl.kernel,
    out_type=jax.ShapeDtypeStruct((NC * NPAD, DEGW), jnp.float32),
    mesh=_sc_mesh,
    scratch_types=[
        pltpu.VMEM((NCH, K), jnp.int32),
        pltpu.VMEM((K, DEGW), jnp.float32),
        pltpu.SemaphoreType.DMA,
        pltpu.VMEM_SHARED((NPAD, DEGW), jnp.float32),
    ],
    compiler_params=pltpu.CompilerParams(use_tc_tiling_on_sc=False),
)
def _deg_kernel(dst_hbm, z16_hbm, out_hbm, dst2_v, ones_v, ssem, acc_sh):
    c = lax.axis_index("c")
    s = lax.axis_index("s")
    w = c * NS + s
    pltpu.sync_copy(dst_hbm.at[w], dst2_v)

    def fill(i, carry):
        ones_v[i, :] = jnp.ones((16,), jnp.float32)
        return carry

    lax.fori_loop(0, K, fill, 0)
    pltpu.sync_copy(z16_hbm.at[pl.ds(s * RPT, RPT)], acc_sh.at[pl.ds(s * RPT, RPT)])
    plsc.subcore_barrier()

    for b in range(WIN):
        pltpu.async_copy(ones_v, acc_sh.at[dst2_v.at[b]], ssem, add=True)

    def chunk(i, carry):
        pltpu.make_async_copy(ones_v, acc_sh.at[dst2_v.at[0]], ssem).wait()
        pltpu.async_copy(ones_v, acc_sh.at[dst2_v.at[i + WIN]], ssem, add=True)
        return carry

    lax.fori_loop(0, NCH - WIN, chunk, 0)
    for b in range(WIN):
        pltpu.make_async_copy(ones_v, acc_sh.at[dst2_v.at[0]], ssem).wait()
    plsc.subcore_barrier()
    pltpu.sync_copy(acc_sh.at[pl.ds(s * RPT, RPT)],
                    out_hbm.at[pl.ds(c * NPAD + s * RPT, RPT)])


@functools.partial(
    pl.kernel,
    out_type=jax.ShapeDtypeStruct((NC * NPAD, H), jnp.float32),
    mesh=_sc_mesh,
    scratch_types=[
        pltpu.VMEM((NCH, K), jnp.int32),
        pltpu.VMEM((NCH, K), jnp.int32),
        pltpu.VMEM((NBUF, K, H), jnp.float32),
        pltpu.SemaphoreType.DMA,
    ] + [pltpu.SemaphoreType.DMA] * NBUF + [
        pltpu.VMEM_SHARED((NPAD, H), jnp.float32),
    ],
    compiler_params=pltpu.CompilerParams(use_tc_tiling_on_sc=False),
)
def _msg_kernel(m_hbm, src_hbm, dst_hbm, out_hbm, src2_v, dst2_v, rows_v, gsem,
                *rest):
    ssems = list(rest[:NBUF])
    acc_sh = rest[NBUF]
    c = lax.axis_index("c")
    s = lax.axis_index("s")
    w = c * NS + s
    pltpu.sync_copy(src_hbm.at[w], src2_v)
    pltpu.sync_copy(dst_hbm.at[w], dst2_v)
    for b in range(AHEAD):
        pltpu.async_copy(m_hbm.at[src2_v.at[b]], rows_v.at[b], gsem)
    pltpu.sync_copy(m_hbm.at[pl.ds(s * RPT, RPT)], acc_sh.at[pl.ds(s * RPT, RPT)])
    plsc.subcore_barrier()

    def step(j, carry, tail):
        for b in (range(NCH % NBUF) if tail else range(NBUF)):
            i = j * NBUF + b
            pltpu.make_async_copy(m_hbm.at[src2_v.at[0]], rows_v.at[b],
                                  gsem).wait()
            pltpu.async_copy(rows_v.at[b], acc_sh.at[dst2_v.at[i]],
                             ssems[b], add=True)
            if not tail:
                bn = (b + AHEAD) % NBUF

                @pl.when(i + AHEAD < NCH)
                def _():
                    @pl.when(i >= NBUF - AHEAD)
                    def _():
                        pltpu.make_async_copy(rows_v.at[bn],
                                              acc_sh.at[dst2_v.at[0]],
                                              ssems[bn]).wait()

                    pltpu.async_copy(m_hbm.at[src2_v.at[i + AHEAD]],
                                     rows_v.at[bn], gsem)
        return carry

    nfull = NCH // NBUF
    lax.fori_loop(0, nfull, functools.partial(step, tail=False), 0)
    if NCH % NBUF:
        step(nfull, 0, tail=True)
    for b in range(NBUF):
        pltpu.make_async_copy(rows_v.at[0], acc_sh.at[dst2_v.at[0]],
                              ssems[b]).wait()
    plsc.subcore_barrier()
    pltpu.sync_copy(acc_sh.at[pl.ds(s * RPT, RPT)],
                    out_hbm.at[pl.ds(c * NPAD + s * RPT, RPT)])


def _dinv(deg_ref):
    deg = 1.0 + deg_ref[0, :, 0:1] + deg_ref[1, :, 0:1]
    return lax.rsqrt(deg)


def _tc_xw1_body(x_ref, w1_ref, o_ref):
    o_ref[...] = jnp.dot(x_ref[...], w1_ref[...],
                         preferred_element_type=jnp.float32)


def _tc_m1_body(deg_ref, xw1_ref, m1_ref):
    m1_ref[...] = xw1_ref[...] * _dinv(deg_ref)


def _tc_mid_body(acc_ref, m1_ref, deg_ref, b1_ref, w2_ref, m2_ref):
    dinv = _dinv(deg_ref)
    tot = acc_ref[0] + acc_ref[1] - m1_ref[...]
    h1 = jnp.maximum(tot * dinv + b1_ref[...], 0.0)
    m2_ref[...] = jnp.dot(h1, w2_ref[...],
                          preferred_element_type=jnp.float32) * dinv


def _tc_final_body(acc_ref, m2_ref, deg_ref, b2_ref, batch_ref, w3_ref,
                   b3_ref, w4_ref, b4_ref, psum_ref, pcnt_ref, o_ref):
    i = pl.program_id(0)
    dinv = _dinv(deg_ref)
    tot = acc_ref[0] + acc_ref[1] - m2_ref[...]
    h2 = jnp.maximum(tot * dinv + b2_ref[...], 0.0)
    b = batch_ref[0, 0, :]
    onehot = (b[:, None] == lax.broadcasted_iota(jnp.int32, (BN, G), 1))
    onehot = onehot.astype(jnp.float32)
    ps = lax.dot_general(onehot, h2, (((0,), (0,)), ((), ())),
                         preferred_element_type=jnp.float32)
    cs = lax.dot_general(onehot, jnp.ones((BN, H), jnp.float32),
                         (((0,), (0,)), ((), ())),
                         preferred_element_type=jnp.float32)

    @pl.when(i == 0)
    def _():
        psum_ref[...] = jnp.zeros_like(psum_ref)
        pcnt_ref[...] = jnp.zeros_like(pcnt_ref)

    psum_ref[...] += ps
    pcnt_ref[...] += cs

    @pl.when(i == NB - 1)
    def _():
        pooled = psum_ref[...] / jnp.maximum(pcnt_ref[...], 1.0)
        t = jnp.maximum(jnp.dot(pooled, w3_ref[...],
                                preferred_element_type=jnp.float32)
                        + b3_ref[...], 0.0)
        o_ref[...] = jnp.dot(t, w4_ref[...],
                             preferred_element_type=jnp.float32) + b4_ref[...]


def _full(shape):
    nd = len(shape)
    return pl.BlockSpec(shape, lambda i: (0,) * nd)


def kernel(x, edge_index, batch, W1, b1, W2, b2, W3, b3, W4, b4):
    x = x.astype(jnp.float32)
    src = jnp.pad(edge_index[0].reshape(NW, EPT),
                  ((0, 0), (0, EPAD))).reshape(NW, NCH, K)
    dst = jnp.pad(edge_index[1].reshape(NW, EPT), ((0, 0), (0, EPAD)),
                  constant_values=NPAD - 1).reshape(NW, NCH, K)
    xp = jnp.pad(x, ((0, NPAD - N), (0, 0)))
    z16 = jnp.zeros((NPAD, DEGW), jnp.float32)

    xw1 = pl.pallas_call(
        _tc_xw1_body,
        grid=(NB,),
        in_specs=[
            pl.BlockSpec((BN, D), lambda i: (i, 0)),
            _full((D, H)),
        ],
        out_specs=pl.BlockSpec((BN, H), lambda i: (i, 0)),
        out_shape=jax.ShapeDtypeStruct((NPAD, H), jnp.float32),
    )(xp, W1)

    degparts = _deg_kernel(dst, z16).reshape(NC, NPAD, DEGW)

    m1 = pl.pallas_call(
        _tc_m1_body,
        grid=(NB,),
        in_specs=[
            pl.BlockSpec((NC, BN, DEGW), lambda i: (0, i, 0)),
            pl.BlockSpec((BN, H), lambda i: (i, 0)),
        ],
        out_specs=pl.BlockSpec((BN, H), lambda i: (i, 0)),
        out_shape=jax.ShapeDtypeStruct((NPAD, H), jnp.float32),
    )(degparts, xw1)

    acc1 = _msg_kernel(m1, src, dst).reshape(NC, NPAD, H)

    m2 = pl.pallas_call(
        _tc_mid_body,
        grid=(NB,),
        in_specs=[
            pl.BlockSpec((NC, BN, H), lambda i: (0, i, 0)),
            pl.BlockSpec((BN, H), lambda i: (i, 0)),
            pl.BlockSpec((NC, BN, DEGW), lambda i: (0, i, 0)),
            _full((1, H)),
            _full((H, H)),
        ],
        out_specs=pl.BlockSpec((BN, H), lambda i: (i, 0)),
        out_shape=jax.ShapeDtypeStruct((NPAD, H), jnp.float32),
    )(acc1, m1, degparts, b1.reshape(1, H), W2)

    acc2 = _msg_kernel(m2, src, dst).reshape(NC, NPAD, H)

    batchp = jnp.pad(batch, (0, NPAD - N), constant_values=G)
    batchp = batchp.reshape(NB, 1, BN)
    w4p = jnp.pad(W4, ((0, 0), (0, 7)))

    _, _, o8 = pl.pallas_call(
        _tc_final_body,
        grid=(NB,),
        in_specs=[
            pl.BlockSpec((NC, BN, H), lambda i: (0, i, 0)),
            pl.BlockSpec((BN, H), lambda i: (i, 0)),
            pl.BlockSpec((NC, BN, DEGW), lambda i: (0, i, 0)),
            _full((1, H)),
            pl.BlockSpec((1, 1, BN), lambda i: (i, 0, 0)),
            _full((H, H)),
            _full((1, H)),
            _full((H, 8)),
            _full((1, 1)),
        ],
        out_specs=[_full((G, H)), _full((G, H)), _full((G, 8))],
        out_shape=[
            jax.ShapeDtypeStruct((G, H), jnp.float32),
            jax.ShapeDtypeStruct((G, H), jnp.float32),
            jax.ShapeDtypeStruct((G, 8), jnp.float32),
        ],
    )(acc2, m2, degparts, b2.reshape(1, H), batchp, W3,
      b3.reshape(1, H), w4p, b4.reshape(1, 1))

    return o8[:, 0]


# speedup vs baseline: 1.7064x; 1.0925x over previous
import functools

import jax
import jax.numpy as jnp
from jax import lax
from jax.experimental import pallas as pl
from jax.experimental.pallas import tpu as pltpu
from jax.experimental.pallas import tpu_sc as plsc

N = 10000
E = 320000
D = 128
H = 64
G = 128

NC = 2
NS = 16
NW = NC * NS
NPAD = 10240
EPC = E // NC
EPT = E // NW
K = 80
NCH = -(-EPT // K)
EPAD = NCH * K - EPT
DEGW = 16
RPT = NPAD // NS

BN = 1024
NB = NPAD // BN

_sc_mesh = plsc.VectorSubcoreMesh(core_axis_name="c", subcore_axis_name="s")


WIN = 16
NBUF = 12
AHEAD = 6


@functools.partial(
    pl.kernel,
    out_type=jax.ShapeDtypeStruct((NC, NPAD, DEGW), jnp.float32),
    mesh=_sc_mesh,
    scratch_types=[
        pltpu.VMEM((NCH, K), jnp.int32),
        pltpu.VMEM((K, DEGW), jnp.float32),
        pltpu.SemaphoreType.DMA,
        pltpu.VMEM_SHARED((NPAD, DEGW), jnp.float32),
    ],
    compiler_params=pltpu.CompilerParams(use_tc_tiling_on_sc=False),
)
def _deg_kernel(dst_hbm, z16_hbm, out_hbm, dst2_v, ones_v, ssem, acc_sh):
    c = lax.axis_index("c")
    s = lax.axis_index("s")
    w = c * NS + s
    pltpu.sync_copy(dst_hbm.at[w], dst2_v)

    def fill(i, carry):
        ones_v[i, :] = jnp.ones((16,), jnp.float32)
        return carry

    lax.fori_loop(0, K, fill, 0)
    pltpu.sync_copy(z16_hbm.at[pl.ds(s * RPT, RPT)], acc_sh.at[pl.ds(s * RPT, RPT)])
    plsc.subcore_barrier()

    for b in range(WIN):
        pltpu.async_copy(ones_v, acc_sh.at[dst2_v.at[b]], ssem, add=True)

    def chunk(i, carry):
        pltpu.make_async_copy(ones_v, acc_sh.at[dst2_v.at[0]], ssem).wait()
        pltpu.async_copy(ones_v, acc_sh.at[dst2_v.at[i + WIN]], ssem, add=True)
        return carry

    lax.fori_loop(0, NCH - WIN, chunk, 0)
    for b in range(WIN):
        pltpu.make_async_copy(ones_v, acc_sh.at[dst2_v.at[0]], ssem).wait()
    plsc.subcore_barrier()
    pltpu.sync_copy(acc_sh.at[pl.ds(s * RPT, RPT)],
                    out_hbm.at[c, pl.ds(s * RPT, RPT)])


@functools.partial(
    pl.kernel,
    out_type=jax.ShapeDtypeStruct((NC, NPAD, H), jnp.float32),
    mesh=_sc_mesh,
    scratch_types=[
        pltpu.VMEM((NCH, K), jnp.int32),
        pltpu.VMEM((NCH, K), jnp.int32),
        pltpu.VMEM((NBUF, K, H), jnp.float32),
        pltpu.SemaphoreType.DMA,
    ] + [pltpu.SemaphoreType.DMA] * NBUF + [
        pltpu.VMEM_SHARED((NPAD, H), jnp.float32),
    ],
    compiler_params=pltpu.CompilerParams(use_tc_tiling_on_sc=False),
)
def _msg_kernel(m_hbm, src_hbm, dst_hbm, out_hbm, src2_v, dst2_v, rows_v, gsem,
                *rest):
    ssems = list(rest[:NBUF])
    acc_sh = rest[NBUF]
    c = lax.axis_index("c")
    s = lax.axis_index("s")
    w = c * NS + s
    pltpu.sync_copy(src_hbm.at[w], src2_v)
    pltpu.sync_copy(dst_hbm.at[w], dst2_v)
    for b in range(AHEAD):
        pltpu.async_copy(m_hbm.at[src2_v.at[b]], rows_v.at[b], gsem)
    pltpu.sync_copy(m_hbm.at[pl.ds(s * RPT, RPT)], acc_sh.at[pl.ds(s * RPT, RPT)])
    plsc.subcore_barrier()

    def step(j, carry, tail):
        for b in (range(NCH % NBUF) if tail else range(NBUF)):
            i = j * NBUF + b
            pltpu.make_async_copy(m_hbm.at[src2_v.at[0]], rows_v.at[b],
                                  gsem).wait()
            pltpu.async_copy(rows_v.at[b], acc_sh.at[dst2_v.at[i]],
                             ssems[b], add=True)
            if not tail:
                bn = (b + AHEAD) % NBUF

                @pl.when(i + AHEAD < NCH)
                def _():
                    @pl.when(i >= NBUF - AHEAD)
                    def _():
                        pltpu.make_async_copy(rows_v.at[bn],
                                              acc_sh.at[dst2_v.at[0]],
                                              ssems[bn]).wait()

                    pltpu.async_copy(m_hbm.at[src2_v.at[i + AHEAD]],
                                     rows_v.at[bn], gsem)
        return carry

    nfull = NCH // NBUF
    lax.fori_loop(0, nfull, functools.partial(step, tail=False), 0)
    if NCH % NBUF:
        step(nfull, 0, tail=True)
    for b in range(NBUF):
        pltpu.make_async_copy(rows_v.at[0], acc_sh.at[dst2_v.at[0]],
                              ssems[b]).wait()
    plsc.subcore_barrier()
    pltpu.sync_copy(acc_sh.at[pl.ds(s * RPT, RPT)],
                    out_hbm.at[c, pl.ds(s * RPT, RPT)])


def _dinv(deg_ref):
    deg = 1.0 + deg_ref[0, :, 0:1] + deg_ref[1, :, 0:1]
    return lax.rsqrt(deg)


def _tc_m1_body(deg_ref, x_ref, w1_ref, m1_ref):
    m1_ref[...] = jnp.dot(x_ref[...], w1_ref[...],
                          preferred_element_type=jnp.float32) * _dinv(deg_ref)


def _tc_mid_body(acc_ref, m1_ref, deg_ref, b1_ref, w2_ref, m2_ref):
    dinv = _dinv(deg_ref)
    tot = acc_ref[0] + acc_ref[1] - m1_ref[...]
    h1 = jnp.maximum(tot * dinv + b1_ref[...], 0.0)
    m2_ref[...] = jnp.dot(h1, w2_ref[...],
                          preferred_element_type=jnp.float32) * dinv


def _tc_final_body(acc_ref, m2_ref, deg_ref, b2_ref, batch_ref, w3_ref,
                   b3_ref, w4_ref, b4_ref, psum_ref, pcnt_ref, o_ref):
    i = pl.program_id(0)
    dinv = _dinv(deg_ref)
    tot = acc_ref[0] + acc_ref[1] - m2_ref[...]
    h2 = jnp.maximum(tot * dinv + b2_ref[...], 0.0)
    b = batch_ref[0, 0, :]
    onehot = (b[:, None] == lax.broadcasted_iota(jnp.int32, (BN, G), 1))
    onehot = onehot.astype(jnp.float32)
    ps = lax.dot_general(onehot, h2, (((0,), (0,)), ((), ())),
                         preferred_element_type=jnp.float32)
    cs = lax.dot_general(onehot, jnp.ones((BN, H), jnp.float32),
                         (((0,), (0,)), ((), ())),
                         preferred_element_type=jnp.float32)

    @pl.when(i == 0)
    def _():
        psum_ref[...] = jnp.zeros_like(psum_ref)
        pcnt_ref[...] = jnp.zeros_like(pcnt_ref)

    psum_ref[...] += ps
    pcnt_ref[...] += cs

    @pl.when(i == NB - 1)
    def _():
        pooled = psum_ref[...] / jnp.maximum(pcnt_ref[...], 1.0)
        t = jnp.maximum(jnp.dot(pooled, w3_ref[...],
                                preferred_element_type=jnp.float32)
                        + b3_ref[...], 0.0)
        o_ref[...] = jnp.dot(t, w4_ref[...],
                             preferred_element_type=jnp.float32) + b4_ref[...]


def _full(shape):
    nd = len(shape)
    return pl.BlockSpec(shape, lambda i: (0,) * nd)


def kernel(x, edge_index, batch, W1, b1, W2, b2, W3, b3, W4, b4):
    x = x.astype(jnp.float32)
    src = jnp.pad(edge_index[0].reshape(NW, EPT),
                  ((0, 0), (0, EPAD))).reshape(NW, NCH, K)
    dst = jnp.pad(edge_index[1].reshape(NW, EPT), ((0, 0), (0, EPAD)),
                  constant_values=NPAD - 1).reshape(NW, NCH, K)
    xp = jnp.pad(x, ((0, NPAD - N), (0, 0)))
    z16 = jnp.zeros((NPAD, DEGW), jnp.float32)

    degparts = _deg_kernel(dst, z16)

    m1 = pl.pallas_call(
        _tc_m1_body,
        grid=(NB,),
        in_specs=[
            pl.BlockSpec((NC, BN, DEGW), lambda i: (0, i, 0)),
            pl.BlockSpec((BN, D), lambda i: (i, 0)),
            _full((D, H)),
        ],
        out_specs=pl.BlockSpec((BN, H), lambda i: (i, 0)),
        out_shape=jax.ShapeDtypeStruct((NPAD, H), jnp.float32),
    )(degparts, xp, W1)

    acc1 = _msg_kernel(m1, src, dst)

    m2 = pl.pallas_call(
        _tc_mid_body,
        grid=(NB,),
        in_specs=[
            pl.BlockSpec((NC, BN, H), lambda i: (0, i, 0)),
            pl.BlockSpec((BN, H), lambda i: (i, 0)),
            pl.BlockSpec((NC, BN, DEGW), lambda i: (0, i, 0)),
            _full((1, H)),
            _full((H, H)),
        ],
        out_specs=pl.BlockSpec((BN, H), lambda i: (i, 0)),
        out_shape=jax.ShapeDtypeStruct((NPAD, H), jnp.float32),
    )(acc1, m1, degparts, b1.reshape(1, H), W2)

    acc2 = _msg_kernel(m2, src, dst)

    batchp = jnp.pad(batch, (0, NPAD - N), constant_values=G)
    batchp = batchp.reshape(NB, 1, BN)
    w4p = jnp.pad(W4, ((0, 0), (0, 7)))

    _, _, o8 = pl.pallas_call(
        _tc_final_body,
        grid=(NB,),
        in_specs=[
            pl.BlockSpec((NC, BN, H), lambda i: (0, i, 0)),
            pl.BlockSpec((BN, H), lambda i: (i, 0)),
            pl.BlockSpec((NC, BN, DEGW), lambda i: (0, i, 0)),
            _full((1, H)),
            pl.BlockSpec((1, 1, BN), lambda i: (i, 0, 0)),
            _full((H, H)),
            _full((1, H)),
            _full((H, 8)),
            _full((1, 1)),
        ],
        out_specs=[_full((G, H)), _full((G, H)), _full((G, 8))],
        out_shape=[
            jax.ShapeDtypeStruct((G, H), jnp.float32),
            jax.ShapeDtypeStruct((G, H), jnp.float32),
            jax.ShapeDtypeStruct((G, 8), jnp.float32),
        ],
    )(acc2, m2, degparts, b2.reshape(1, H), batchp, W3,
      b3.reshape(1, H), w4p, b4.reshape(1, 1))

    return o8[:, 0]


# speedup vs baseline: 1.8586x; 1.0892x over previous
import functools

import jax
import jax.numpy as jnp
from jax import lax
from jax.experimental import pallas as pl
from jax.experimental.pallas import tpu as pltpu
from jax.experimental.pallas import tpu_sc as plsc

N = 10000
E = 320000
D = 128
H = 64
G = 128

NC = 2
NS = 16
NW = NC * NS
NPAD = 10240
EPC = E // NC
EPT = E // NW
K = 80
NCH = -(-EPT // K)
EPAD = NCH * K - EPT
DEGW = 16
RPT = NPAD // NS

BN = 2048
NB = NPAD // BN

_sc_mesh = plsc.VectorSubcoreMesh(core_axis_name="c", subcore_axis_name="s")


WIN = 16
NBUF = 12
AHEAD = 6


@functools.partial(
    pl.kernel,
    out_type=jax.ShapeDtypeStruct((NC, NPAD, DEGW), jnp.float32),
    mesh=_sc_mesh,
    scratch_types=[
        pltpu.VMEM((NCH, K), jnp.int32),
        pltpu.VMEM((K, DEGW), jnp.float32),
        pltpu.SemaphoreType.DMA,
        pltpu.VMEM_SHARED((NPAD, DEGW), jnp.float32),
    ],
    compiler_params=pltpu.CompilerParams(use_tc_tiling_on_sc=False),
)
def _deg_kernel(ei_hbm, z16_hbm, out_hbm, dst2_v, ones_v, ssem, acc_sh):
    c = lax.axis_index("c")
    s = lax.axis_index("s")
    w = c * NS + s
    pltpu.sync_copy(ei_hbm.at[1, w], dst2_v)

    def fill(i, carry):
        ones_v[i, :] = jnp.ones((16,), jnp.float32)
        return carry

    lax.fori_loop(0, K, fill, 0)
    pltpu.sync_copy(z16_hbm.at[pl.ds(s * RPT, RPT)], acc_sh.at[pl.ds(s * RPT, RPT)])
    plsc.subcore_barrier()

    for b in range(WIN):
        pltpu.async_copy(ones_v, acc_sh.at[dst2_v.at[b]], ssem, add=True)

    def chunk(i, carry):
        pltpu.make_async_copy(ones_v, acc_sh.at[dst2_v.at[0]], ssem).wait()
        pltpu.async_copy(ones_v, acc_sh.at[dst2_v.at[i + WIN]], ssem, add=True)
        return carry

    lax.fori_loop(0, NCH - WIN, chunk, 0)
    for b in range(WIN):
        pltpu.make_async_copy(ones_v, acc_sh.at[dst2_v.at[0]], ssem).wait()
    plsc.subcore_barrier()
    pltpu.sync_copy(acc_sh.at[pl.ds(s * RPT, RPT)],
                    out_hbm.at[c, pl.ds(s * RPT, RPT)])


@functools.partial(
    pl.kernel,
    out_type=jax.ShapeDtypeStruct((NC, NPAD, H), jnp.float32),
    mesh=_sc_mesh,
    scratch_types=[
        pltpu.VMEM((NCH, K), jnp.int32),
        pltpu.VMEM((NCH, K), jnp.int32),
        pltpu.VMEM((NBUF, K, H), jnp.float32),
        pltpu.SemaphoreType.DMA,
    ] + [pltpu.SemaphoreType.DMA] * NBUF + [
        pltpu.VMEM_SHARED((NPAD, H), jnp.float32),
    ],
    compiler_params=pltpu.CompilerParams(use_tc_tiling_on_sc=False),
)
def _msg_kernel(m_hbm, ei_hbm, out_hbm, src2_v, dst2_v, rows_v, gsem,
                *rest):
    ssems = list(rest[:NBUF])
    acc_sh = rest[NBUF]
    c = lax.axis_index("c")
    s = lax.axis_index("s")
    w = c * NS + s
    pltpu.sync_copy(ei_hbm.at[0, w], src2_v)
    pltpu.sync_copy(ei_hbm.at[1, w], dst2_v)
    for b in range(AHEAD):
        pltpu.async_copy(m_hbm.at[src2_v.at[b]], rows_v.at[b], gsem)
    pltpu.sync_copy(m_hbm.at[pl.ds(s * RPT, RPT)], acc_sh.at[pl.ds(s * RPT, RPT)])
    plsc.subcore_barrier()

    def step(j, carry, tail):
        for b in (range(NCH % NBUF) if tail else range(NBUF)):
            i = j * NBUF + b
            pltpu.make_async_copy(m_hbm.at[src2_v.at[0]], rows_v.at[b],
                                  gsem).wait()
            pltpu.async_copy(rows_v.at[b], acc_sh.at[dst2_v.at[i]],
                             ssems[b], add=True)
            if not tail:
                bn = (b + AHEAD) % NBUF

                @pl.when(i + AHEAD < NCH)
                def _():
                    @pl.when(i >= NBUF - AHEAD)
                    def _():
                        pltpu.make_async_copy(rows_v.at[bn],
                                              acc_sh.at[dst2_v.at[0]],
                                              ssems[bn]).wait()

                    pltpu.async_copy(m_hbm.at[src2_v.at[i + AHEAD]],
                                     rows_v.at[bn], gsem)
        return carry

    nfull = NCH // NBUF
    lax.fori_loop(0, nfull, functools.partial(step, tail=False), 0)
    if NCH % NBUF:
        step(nfull, 0, tail=True)
    for b in range(NBUF):
        pltpu.make_async_copy(rows_v.at[0], acc_sh.at[dst2_v.at[0]],
                              ssems[b]).wait()
    plsc.subcore_barrier()
    pltpu.sync_copy(acc_sh.at[pl.ds(s * RPT, RPT)],
                    out_hbm.at[c, pl.ds(s * RPT, RPT)])


def _dinv(deg_ref):
    deg = 1.0 + deg_ref[0, :, 0:1] + deg_ref[1, :, 0:1]
    return lax.rsqrt(deg)


def _tc_m1_body(deg_ref, x_ref, w1_ref, m1_ref):
    m1_ref[...] = jnp.dot(x_ref[...], w1_ref[...],
                          preferred_element_type=jnp.float32) * _dinv(deg_ref)


def _tc_mid_body(acc_ref, m1_ref, deg_ref, b1_ref, w2_ref, m2_ref):
    dinv = _dinv(deg_ref)
    tot = acc_ref[0] + acc_ref[1] - m1_ref[...]
    h1 = jnp.maximum(tot * dinv + b1_ref[...], 0.0)
    m2_ref[...] = jnp.dot(h1, w2_ref[...],
                          preferred_element_type=jnp.float32) * dinv


def _tc_final_body(acc_ref, m2_ref, deg_ref, b2_ref, batch_ref, w3_ref,
                   b3_ref, w4_ref, b4_ref, psum_ref, pcnt_ref, o_ref):
    i = pl.program_id(0)
    dinv = _dinv(deg_ref)
    tot = acc_ref[0] + acc_ref[1] - m2_ref[...]
    h2 = jnp.maximum(tot * dinv + b2_ref[...], 0.0)
    b = batch_ref[0, 0, :]
    onehot = (b[:, None] == lax.broadcasted_iota(jnp.int32, (BN, G), 1))
    onehot = onehot.astype(jnp.float32)
    ps = lax.dot_general(onehot, h2, (((0,), (0,)), ((), ())),
                         preferred_element_type=jnp.float32)
    cs = lax.dot_general(onehot, jnp.ones((BN, H), jnp.float32),
                         (((0,), (0,)), ((), ())),
                         preferred_element_type=jnp.float32)

    @pl.when(i == 0)
    def _():
        psum_ref[...] = jnp.zeros_like(psum_ref)
        pcnt_ref[...] = jnp.zeros_like(pcnt_ref)

    psum_ref[...] += ps
    pcnt_ref[...] += cs

    @pl.when(i == NB - 1)
    def _():
        pooled = psum_ref[...] / jnp.maximum(pcnt_ref[...], 1.0)
        t = jnp.maximum(jnp.dot(pooled, w3_ref[...],
                                preferred_element_type=jnp.float32)
                        + b3_ref[...], 0.0)
        o_ref[...] = jnp.dot(t, w4_ref[...],
                             preferred_element_type=jnp.float32) + b4_ref[...]


def _full(shape):
    nd = len(shape)
    return pl.BlockSpec(shape, lambda i: (0,) * nd)


def kernel(x, edge_index, batch, W1, b1, W2, b2, W3, b3, W4, b4):
    x = x.astype(jnp.float32)
    ei = edge_index.reshape(2, NW, NCH, K)
    xp = jnp.pad(x, ((0, NPAD - N), (0, 0)))
    z16 = jnp.zeros((NPAD, DEGW), jnp.float32)

    degparts = _deg_kernel(ei, z16)

    m1 = pl.pallas_call(
        _tc_m1_body,
        grid=(NB,),
        in_specs=[
            pl.BlockSpec((NC, BN, DEGW), lambda i: (0, i, 0)),
            pl.BlockSpec((BN, D), lambda i: (i, 0)),
            _full((D, H)),
        ],
        out_specs=pl.BlockSpec((BN, H), lambda i: (i, 0)),
        out_shape=jax.ShapeDtypeStruct((NPAD, H), jnp.float32),
    )(degparts, xp, W1)

    acc1 = _msg_kernel(m1, ei)

    m2 = pl.pallas_call(
        _tc_mid_body,
        grid=(NB,),
        in_specs=[
            pl.BlockSpec((NC, BN, H), lambda i: (0, i, 0)),
            pl.BlockSpec((BN, H), lambda i: (i, 0)),
            pl.BlockSpec((NC, BN, DEGW), lambda i: (0, i, 0)),
            _full((1, H)),
            _full((H, H)),
        ],
        out_specs=pl.BlockSpec((BN, H), lambda i: (i, 0)),
        out_shape=jax.ShapeDtypeStruct((NPAD, H), jnp.float32),
    )(acc1, m1, degparts, b1.reshape(1, H), W2)

    acc2 = _msg_kernel(m2, ei)

    batchp = jnp.pad(batch, (0, NPAD - N), constant_values=G)
    batchp = batchp.reshape(NB, 1, BN)
    w4p = jnp.pad(W4, ((0, 0), (0, 7)))

    _, _, o8 = pl.pallas_call(
        _tc_final_body,
        grid=(NB,),
        in_specs=[
            pl.BlockSpec((NC, BN, H), lambda i: (0, i, 0)),
            pl.BlockSpec((BN, H), lambda i: (i, 0)),
            pl.BlockSpec((NC, BN, DEGW), lambda i: (0, i, 0)),
            _full((1, H)),
            pl.BlockSpec((1, 1, BN), lambda i: (i, 0, 0)),
            _full((H, H)),
            _full((1, H)),
            _full((H, 8)),
            _full((1, 1)),
        ],
        out_specs=[_full((G, H)), _full((G, H)), _full((G, 8))],
        out_shape=[
            jax.ShapeDtypeStruct((G, H), jnp.float32),
            jax.ShapeDtypeStruct((G, H), jnp.float32),
            jax.ShapeDtypeStruct((G, 8), jnp.float32),
        ],
    )(acc2, m2, degparts, b2.reshape(1, H), batchp, W3,
      b3.reshape(1, H), w4p, b4.reshape(1, 1))

    return o8[:, 0]


# speedup vs baseline: 2.0441x; 1.0998x over previous
import functools

import jax
import jax.numpy as jnp
from jax import lax
from jax.experimental import pallas as pl
from jax.experimental.pallas import tpu as pltpu
from jax.experimental.pallas import tpu_sc as plsc

N = 10000
E = 320000
D = 128
H = 64
G = 128

NC = 2
NS = 16
NW = NC * NS
NPAD = 10240
EPC = E // NC
EPT = E // NW
K = 80
NCH = -(-EPT // K)
EPAD = NCH * K - EPT
DEGW = 16
RPT = NPAD // NS

BN = 2048
NB = NPAD // BN

_sc_mesh = plsc.VectorSubcoreMesh(core_axis_name="c", subcore_axis_name="s")


WIN = 16
NBUF = 12
AHEAD = 6


@functools.partial(
    pl.kernel,
    out_type=jax.ShapeDtypeStruct((NC, NPAD, DEGW), jnp.float32),
    mesh=_sc_mesh,
    scratch_types=[
        pltpu.VMEM((NCH, K), jnp.int32),
        pltpu.VMEM((K, DEGW), jnp.float32),
        pltpu.SemaphoreType.DMA,
        pltpu.VMEM_SHARED((NPAD, DEGW), jnp.float32),
    ],
    compiler_params=pltpu.CompilerParams(use_tc_tiling_on_sc=False),
)
def _deg_kernel(ei_hbm, z16_hbm, out_hbm, dst2_v, ones_v, ssem, acc_sh):
    c = lax.axis_index("c")
    s = lax.axis_index("s")
    w = c * NS + s
    pltpu.sync_copy(ei_hbm.at[1, w], dst2_v)

    def fill(i, carry):
        ones_v[i, :] = jnp.ones((16,), jnp.float32)
        return carry

    lax.fori_loop(0, K, fill, 0)
    pltpu.sync_copy(z16_hbm.at[pl.ds(s * RPT, RPT)], acc_sh.at[pl.ds(s * RPT, RPT)])
    plsc.subcore_barrier()

    for b in range(WIN):
        pltpu.async_copy(ones_v, acc_sh.at[dst2_v.at[b]], ssem, add=True)

    def chunk(i, carry):
        pltpu.make_async_copy(ones_v, acc_sh.at[dst2_v.at[0]], ssem).wait()
        pltpu.async_copy(ones_v, acc_sh.at[dst2_v.at[i + WIN]], ssem, add=True)
        return carry

    lax.fori_loop(0, NCH - WIN, chunk, 0)
    for b in range(WIN):
        pltpu.make_async_copy(ones_v, acc_sh.at[dst2_v.at[0]], ssem).wait()
    plsc.subcore_barrier()
    pltpu.sync_copy(acc_sh.at[pl.ds(s * RPT, RPT)],
                    out_hbm.at[c, pl.ds(s * RPT, RPT)])


@functools.partial(
    pl.kernel,
    out_type=jax.ShapeDtypeStruct((NPAD, NC * H), jnp.float32),
    mesh=_sc_mesh,
    scratch_types=[
        pltpu.VMEM((NCH, K), jnp.int32),
        pltpu.VMEM((NCH, K), jnp.int32),
        pltpu.VMEM((NBUF, K, H), jnp.float32),
        pltpu.SemaphoreType.DMA,
    ] + [pltpu.SemaphoreType.DMA] * NBUF + [
        pltpu.VMEM_SHARED((NPAD, H), jnp.float32),
    ],
    compiler_params=pltpu.CompilerParams(use_tc_tiling_on_sc=False),
)
def _msg_kernel(m_hbm, ei_hbm, out_hbm, src2_v, dst2_v, rows_v, gsem,
                *rest):
    ssems = list(rest[:NBUF])
    acc_sh = rest[NBUF]
    c = lax.axis_index("c")
    s = lax.axis_index("s")
    w = c * NS + s
    pltpu.sync_copy(ei_hbm.at[0, w], src2_v)
    pltpu.sync_copy(ei_hbm.at[1, w], dst2_v)
    for b in range(AHEAD):
        pltpu.async_copy(m_hbm.at[src2_v.at[b]], rows_v.at[b], gsem)
    pltpu.sync_copy(m_hbm.at[pl.ds(s * RPT, RPT)], acc_sh.at[pl.ds(s * RPT, RPT)])
    plsc.subcore_barrier()

    def step(j, carry, tail):
        for b in (range(NCH % NBUF) if tail else range(NBUF)):
            i = j * NBUF + b
            pltpu.make_async_copy(m_hbm.at[src2_v.at[0]], rows_v.at[b],
                                  gsem).wait()
            pltpu.async_copy(rows_v.at[b], acc_sh.at[dst2_v.at[i]],
                             ssems[b], add=True)
            if not tail:
                bn = (b + AHEAD) % NBUF

                @pl.when(i + AHEAD < NCH)
                def _():
                    @pl.when(i >= NBUF - AHEAD)
                    def _():
                        pltpu.make_async_copy(rows_v.at[bn],
                                              acc_sh.at[dst2_v.at[0]],
                                              ssems[bn]).wait()

                    pltpu.async_copy(m_hbm.at[src2_v.at[i + AHEAD]],
                                     rows_v.at[bn], gsem)
        return carry

    nfull = NCH // NBUF
    lax.fori_loop(0, nfull, functools.partial(step, tail=False), 0)
    if NCH % NBUF:
        step(nfull, 0, tail=True)
    for b in range(NBUF):
        pltpu.make_async_copy(rows_v.at[0], acc_sh.at[dst2_v.at[0]],
                              ssems[b]).wait()
    plsc.subcore_barrier()
    pltpu.sync_copy(acc_sh.at[pl.ds(s * RPT, RPT)],
                    out_hbm.at[pl.ds(s * RPT, RPT), pl.ds(c * H, H)])


def _dinv(deg_ref):
    deg = 1.0 + deg_ref[0, :, 0:1] + deg_ref[1, :, 0:1]
    return lax.rsqrt(deg)


def _tc_m1_body(deg_ref, x_ref, w1_ref, m1_ref):
    m1_ref[...] = jnp.dot(x_ref[...], w1_ref[...],
                          preferred_element_type=jnp.float32) * _dinv(deg_ref)


def _tc_mid_body(acc_ref, m1_ref, deg_ref, b1_ref, w2_ref, m2_ref):
    dinv = _dinv(deg_ref)
    tot = acc_ref[:, 0:H] + acc_ref[:, H:2 * H] - m1_ref[...]
    h1 = jnp.maximum(tot * dinv + b1_ref[...], 0.0)
    m2_ref[...] = jnp.dot(h1, w2_ref[...],
                          preferred_element_type=jnp.float32) * dinv


def _tc_final_body(acc_ref, m2_ref, deg_ref, b2_ref, batch_ref, w3_ref,
                   b3_ref, w4_ref, b4_ref, psum_ref, pcnt_ref, o_ref):
    i = pl.program_id(0)
    dinv = _dinv(deg_ref)
    tot = acc_ref[:, 0:H] + acc_ref[:, H:2 * H] - m2_ref[...]
    h2 = jnp.maximum(tot * dinv + b2_ref[...], 0.0)
    b = batch_ref[0, 0, :]
    onehot = (b[:, None] == lax.broadcasted_iota(jnp.int32, (BN, G), 1))
    onehot = onehot.astype(jnp.float32)
    ps = lax.dot_general(onehot, h2, (((0,), (0,)), ((), ())),
                         preferred_element_type=jnp.float32)
    cs = lax.dot_general(onehot, jnp.ones((BN, H), jnp.float32),
                         (((0,), (0,)), ((), ())),
                         preferred_element_type=jnp.float32)

    @pl.when(i == 0)
    def _():
        psum_ref[...] = jnp.zeros_like(psum_ref)
        pcnt_ref[...] = jnp.zeros_like(pcnt_ref)

    psum_ref[...] += ps
    pcnt_ref[...] += cs

    @pl.when(i == NB - 1)
    def _():
        pooled = psum_ref[...] / jnp.maximum(pcnt_ref[...], 1.0)
        t = jnp.maximum(jnp.dot(pooled, w3_ref[...],
                                preferred_element_type=jnp.float32)
                        + b3_ref[...], 0.0)
        o_ref[...] = jnp.dot(t, w4_ref[...],
                             preferred_element_type=jnp.float32) + b4_ref[...]


def _full(shape):
    nd = len(shape)
    return pl.BlockSpec(shape, lambda i: (0,) * nd)


def kernel(x, edge_index, batch, W1, b1, W2, b2, W3, b3, W4, b4):
    x = x.astype(jnp.float32)
    ei = edge_index.reshape(2, NW, NCH, K)
    xp = jnp.pad(x, ((0, NPAD - N), (0, 0)))
    z16 = jnp.zeros((NPAD, DEGW), jnp.float32)

    degparts = _deg_kernel(ei, z16)

    m1 = pl.pallas_call(
        _tc_m1_body,
        grid=(NB,),
        in_specs=[
            pl.BlockSpec((NC, BN, DEGW), lambda i: (0, i, 0)),
            pl.BlockSpec((BN, D), lambda i: (i, 0)),
            _full((D, H)),
        ],
        out_specs=pl.BlockSpec((BN, H), lambda i: (i, 0)),
        out_shape=jax.ShapeDtypeStruct((NPAD, H), jnp.float32),
    )(degparts, xp, W1)

    acc1 = _msg_kernel(m1, ei)

    m2 = pl.pallas_call(
        _tc_mid_body,
        grid=(NB,),
        in_specs=[
            pl.BlockSpec((BN, NC * H), lambda i: (i, 0)),
            pl.BlockSpec((BN, H), lambda i: (i, 0)),
            pl.BlockSpec((NC, BN, DEGW), lambda i: (0, i, 0)),
            _full((1, H)),
            _full((H, H)),
        ],
        out_specs=pl.BlockSpec((BN, H), lambda i: (i, 0)),
        out_shape=jax.ShapeDtypeStruct((NPAD, H), jnp.float32),
    )(acc1, m1, degparts, b1.reshape(1, H), W2)

    acc2 = _msg_kernel(m2, ei)

    batchp = jnp.pad(batch, (0, NPAD - N), constant_values=G)
    batchp = batchp.reshape(NB, 1, BN)
    w4p = jnp.pad(W4, ((0, 0), (0, 7)))

    _, _, o8 = pl.pallas_call(
        _tc_final_body,
        grid=(NB,),
        in_specs=[
            pl.BlockSpec((BN, NC * H), lambda i: (i, 0)),
            pl.BlockSpec((BN, H), lambda i: (i, 0)),
            pl.BlockSpec((NC, BN, DEGW), lambda i: (0, i, 0)),
            _full((1, H)),
            pl.BlockSpec((1, 1, BN), lambda i: (i, 0, 0)),
            _full((H, H)),
            _full((1, H)),
            _full((H, 8)),
            _full((1, 1)),
        ],
        out_specs=[_full((G, H)), _full((G, H)), _full((G, 8))],
        out_shape=[
            jax.ShapeDtypeStruct((G, H), jnp.float32),
            jax.ShapeDtypeStruct((G, H), jnp.float32),
            jax.ShapeDtypeStruct((G, 8), jnp.float32),
        ],
    )(acc2, m2, degparts, b2.reshape(1, H), batchp, W3,
      b3.reshape(1, H), w4p, b4.reshape(1, 1))

    return o8[:, 0]


# speedup vs baseline: 2.0451x; 1.0005x over previous
import functools

import jax
import jax.numpy as jnp
from jax import lax
from jax.experimental import pallas as pl
from jax.experimental.pallas import tpu as pltpu
from jax.experimental.pallas import tpu_sc as plsc

N = 10000
E = 320000
D = 128
H = 64
G = 128

NC = 2
NS = 16
NW = NC * NS
NPAD = 10240
EPC = E // NC
EPT = E // NW
K = 80
NCH = -(-EPT // K)
EPAD = NCH * K - EPT
DEGW = 16
RPT = NPAD // NS

BN = 2048
NB = NPAD // BN

_sc_mesh = plsc.VectorSubcoreMesh(core_axis_name="c", subcore_axis_name="s")


WIN = 16
NBUF = 12
AHEAD = 6


@functools.partial(
    pl.kernel,
    out_type=jax.ShapeDtypeStruct((NC, NPAD, DEGW), jnp.float32),
    mesh=_sc_mesh,
    scratch_types=[
        pltpu.VMEM((NCH, K), jnp.int32),
        pltpu.VMEM((K, DEGW), jnp.float32),
        pltpu.SemaphoreType.DMA,
        pltpu.VMEM_SHARED((NPAD, DEGW), jnp.float32),
    ],
    compiler_params=pltpu.CompilerParams(use_tc_tiling_on_sc=False),
)
def _deg_kernel(ei_hbm, z16_hbm, out_hbm, dst2_v, ones_v, ssem, acc_sh):
    c = lax.axis_index("c")
    s = lax.axis_index("s")
    w = c * NS + s
    pltpu.sync_copy(ei_hbm.at[1, w], dst2_v)

    def fill(i, carry):
        ones_v[i, :] = jnp.ones((16,), jnp.float32)
        return carry

    lax.fori_loop(0, K, fill, 0)
    pltpu.sync_copy(z16_hbm.at[pl.ds(s * RPT, RPT)], acc_sh.at[pl.ds(s * RPT, RPT)])
    plsc.subcore_barrier()

    for b in range(WIN):
        pltpu.async_copy(ones_v, acc_sh.at[dst2_v.at[b]], ssem, add=True)

    def chunk(i, carry):
        pltpu.make_async_copy(ones_v, acc_sh.at[dst2_v.at[0]], ssem).wait()
        pltpu.async_copy(ones_v, acc_sh.at[dst2_v.at[i + WIN]], ssem, add=True)
        return carry

    lax.fori_loop(0, NCH - WIN, chunk, 0)
    for b in range(WIN):
        pltpu.make_async_copy(ones_v, acc_sh.at[dst2_v.at[0]], ssem).wait()
    plsc.subcore_barrier()
    pltpu.sync_copy(acc_sh.at[pl.ds(s * RPT, RPT)],
                    out_hbm.at[c, pl.ds(s * RPT, RPT)])


@functools.partial(
    pl.kernel,
    out_type=jax.ShapeDtypeStruct((NPAD, NC * H), jnp.float32),
    mesh=_sc_mesh,
    scratch_types=[
        pltpu.VMEM((NCH, K), jnp.int32),
        pltpu.VMEM((NCH, K), jnp.int32),
        pltpu.VMEM((NBUF, K, H), jnp.float32),
        pltpu.SemaphoreType.DMA,
    ] + [pltpu.SemaphoreType.DMA] * NBUF + [
        pltpu.VMEM_SHARED((NPAD, H), jnp.float32),
    ],
    compiler_params=pltpu.CompilerParams(use_tc_tiling_on_sc=False),
)
def _msg_kernel(m_hbm, ei_hbm, out_hbm, src2_v, dst2_v, rows_v, gsem,
                *rest):
    ssems = list(rest[:NBUF])
    acc_sh = rest[NBUF]
    c = lax.axis_index("c")
    s = lax.axis_index("s")
    w = c * NS + s
    pltpu.sync_copy(ei_hbm.at[0, w], src2_v)
    pltpu.sync_copy(ei_hbm.at[1, w], dst2_v)
    for b in range(AHEAD):
        pltpu.async_copy(m_hbm.at[src2_v.at[b]], rows_v.at[b], gsem)
    pltpu.sync_copy(m_hbm.at[pl.ds(s * RPT, RPT)], acc_sh.at[pl.ds(s * RPT, RPT)])
    plsc.subcore_barrier()

    def step(j, carry, tail):
        for b in (range(NCH % NBUF) if tail else range(NBUF)):
            i = j * NBUF + b
            pltpu.make_async_copy(m_hbm.at[src2_v.at[0]], rows_v.at[b],
                                  gsem).wait()
            pltpu.async_copy(rows_v.at[b], acc_sh.at[dst2_v.at[i]],
                             ssems[b], add=True)
            if not tail:
                bn = (b + AHEAD) % NBUF

                @pl.when(i + AHEAD < NCH)
                def _():
                    @pl.when(i >= NBUF - AHEAD)
                    def _():
                        pltpu.make_async_copy(rows_v.at[bn],
                                              acc_sh.at[dst2_v.at[0]],
                                              ssems[bn]).wait()

                    pltpu.async_copy(m_hbm.at[src2_v.at[i + AHEAD]],
                                     rows_v.at[bn], gsem)
        return carry

    nfull = NCH // NBUF
    lax.fori_loop(0, nfull, functools.partial(step, tail=False), 0)
    if NCH % NBUF:
        step(nfull, 0, tail=True)
    for b in range(NBUF):
        pltpu.make_async_copy(rows_v.at[0], acc_sh.at[dst2_v.at[0]],
                              ssems[b]).wait()
    plsc.subcore_barrier()

    @pl.when(c == 0)
    def _():
        pltpu.sync_copy(acc_sh.at[pl.ds(s * RPT, RPT)],
                        out_hbm.at[pl.ds(s * RPT, RPT), pl.ds(0, H)])

    @pl.when(c == 1)
    def _():
        pltpu.sync_copy(acc_sh.at[pl.ds(s * RPT, RPT)],
                        out_hbm.at[pl.ds(s * RPT, RPT), pl.ds(H, H)])


def _dinv(deg_ref):
    deg = 1.0 + deg_ref[0, :, 0:1] + deg_ref[1, :, 0:1]
    return lax.rsqrt(deg)


def _tc_m1_body(deg_ref, x_ref, w1_ref, m1_ref):
    m1_ref[...] = jnp.dot(x_ref[...], w1_ref[...],
                          preferred_element_type=jnp.float32) * _dinv(deg_ref)


def _tc_mid_body(acc_ref, m1_ref, deg_ref, b1_ref, w2_ref, m2_ref):
    dinv = _dinv(deg_ref)
    tot = acc_ref[:, 0:H] + acc_ref[:, H:2 * H] - m1_ref[...]
    h1 = jnp.maximum(tot * dinv + b1_ref[...], 0.0)
    m2_ref[...] = jnp.dot(h1, w2_ref[...],
                          preferred_element_type=jnp.float32) * dinv


def _tc_final_body(acc_ref, m2_ref, deg_ref, b2_ref, batch_ref, w3_ref,
                   b3_ref, w4_ref, b4_ref, psum_ref, pcnt_ref, o_ref):
    i = pl.program_id(0)
    dinv = _dinv(deg_ref)
    tot = acc_ref[:, 0:H] + acc_ref[:, H:2 * H] - m2_ref[...]
    h2 = jnp.maximum(tot * dinv + b2_ref[...], 0.0)
    b = batch_ref[0, 0, :]
    onehot = (b[:, None] == lax.broadcasted_iota(jnp.int32, (BN, G), 1))
    onehot = onehot.astype(jnp.float32)
    ps = lax.dot_general(onehot, h2, (((0,), (0,)), ((), ())),
                         preferred_element_type=jnp.float32)
    cs = lax.dot_general(onehot, jnp.ones((BN, H), jnp.float32),
                         (((0,), (0,)), ((), ())),
                         preferred_element_type=jnp.float32)

    @pl.when(i == 0)
    def _():
        psum_ref[...] = jnp.zeros_like(psum_ref)
        pcnt_ref[...] = jnp.zeros_like(pcnt_ref)

    psum_ref[...] += ps
    pcnt_ref[...] += cs

    @pl.when(i == NB - 1)
    def _():
        pooled = psum_ref[...] / jnp.maximum(pcnt_ref[...], 1.0)
        t = jnp.maximum(jnp.dot(pooled, w3_ref[...],
                                preferred_element_type=jnp.float32)
                        + b3_ref[...], 0.0)
        o_ref[...] = jnp.dot(t, w4_ref[...],
                             preferred_element_type=jnp.float32) + b4_ref[...]


def _full(shape):
    nd = len(shape)
    return pl.BlockSpec(shape, lambda i: (0,) * nd)


def kernel(x, edge_index, batch, W1, b1, W2, b2, W3, b3, W4, b4):
    x = x.astype(jnp.float32)
    ei = edge_index.reshape(2, NW, NCH, K)
    xp = jnp.pad(x, ((0, NPAD - N), (0, 0)))
    z16 = jnp.zeros((NPAD, DEGW), jnp.float32)

    degparts = _deg_kernel(ei, z16)

    m1 = pl.pallas_call(
        _tc_m1_body,
        grid=(NB,),
        in_specs=[
            pl.BlockSpec((NC, BN, DEGW), lambda i: (0, i, 0)),
            pl.BlockSpec((BN, D), lambda i: (i, 0)),
            _full((D, H)),
        ],
        out_specs=pl.BlockSpec((BN, H), lambda i: (i, 0)),
        out_shape=jax.ShapeDtypeStruct((NPAD, H), jnp.float32),
    )(degparts, xp, W1)

    acc1 = _msg_kernel(m1, ei)

    m2 = pl.pallas_call(
        _tc_mid_body,
        grid=(NB,),
        in_specs=[
            pl.BlockSpec((BN, NC * H), lambda i: (i, 0)),
            pl.BlockSpec((BN, H), lambda i: (i, 0)),
            pl.BlockSpec((NC, BN, DEGW), lambda i: (0, i, 0)),
            _full((1, H)),
            _full((H, H)),
        ],
        out_specs=pl.BlockSpec((BN, H), lambda i: (i, 0)),
        out_shape=jax.ShapeDtypeStruct((NPAD, H), jnp.float32),
    )(acc1, m1, degparts, b1.reshape(1, H), W2)

    acc2 = _msg_kernel(m2, ei)

    batchp = jnp.pad(batch, (0, NPAD - N), constant_values=G)
    batchp = batchp.reshape(NB, 1, BN)
    w4p = jnp.pad(W4, ((0, 0), (0, 7)))

    _, _, o8 = pl.pallas_call(
        _tc_final_body,
        grid=(NB,),
        in_specs=[
            pl.BlockSpec((BN, NC * H), lambda i: (i, 0)),
            pl.BlockSpec((BN, H), lambda i: (i, 0)),
            pl.BlockSpec((NC, BN, DEGW), lambda i: (0, i, 0)),
            _full((1, H)),
            pl.BlockSpec((1, 1, BN), lambda i: (i, 0, 0)),
            _full((H, H)),
            _full((1, H)),
            _full((H, 8)),
            _full((1, 1)),
        ],
        out_specs=[_full((G, H)), _full((G, H)), _full((G, 8))],
        out_shape=[
            jax.ShapeDtypeStruct((G, H), jnp.float32),
            jax.ShapeDtypeStruct((G, H), jnp.float32),
            jax.ShapeDtypeStruct((G, 8), jnp.float32),
        ],
    )(acc2, m2, degparts, b2.reshape(1, H), batchp, W3,
      b3.reshape(1, H), w4p, b4.reshape(1, 1))

    return o8[:, 0]
